# Initial kernel scaffold; baseline (speedup 1.0000x reference)
import jax
import jax.numpy as jnp
from jax.experimental import pallas as pl


def kernel(input_ids, embed_table):
    raise NotImplementedError("write your pallas kernel here")


# speedup vs baseline: 3.4640x; 3.4640x over previous
import functools

import jax
import jax.numpy as jnp
from jax import lax
from jax.experimental import pallas as pl
from jax.experimental.pallas import tpu as pltpu
from jax.experimental.pallas import tpu_sc as plsc

D = 128
BATCH = 16384
HIST = 50
TOT = BATCH * HIST
NW = 32
GROUP = 128
GPW = TOT // (NW * GROUP)
ROWS_PER_W = TOT // NW
SC_ROWS = 2 * GROUP
NSC = GPW // 2

_mesh = plsc.VectorSubcoreMesh(core_axis_name="c", subcore_axis_name="s")


@functools.partial(
    pl.kernel,
    out_type=jax.ShapeDtypeStruct((TOT, D), jnp.float32),
    mesh=_mesh,
    scratch_types=[
        pltpu.VMEM((GPW, GROUP), jnp.int32),
        pltpu.VMEM((SC_ROWS, D), jnp.float32),
        pltpu.VMEM((SC_ROWS, D), jnp.float32),
        pltpu.SemaphoreType.DMA,
        pltpu.SemaphoreType.DMA,
        pltpu.SemaphoreType.DMA,
        pltpu.SemaphoreType.DMA,
    ],
)
def _embed_gather(table, idx, out, idx_v, buf0, buf1, gsem0, gsem1, wsem0, wsem1):
    wid = lax.axis_index("s") * 2 + lax.axis_index("c")
    base = wid * ROWS_PER_W

    pltpu.sync_copy(idx.at[pl.ds(wid * GPW, GPW)], idx_v)

    bufs = (buf0, buf1)
    gsems = (gsem0, gsem1)
    wsems = (wsem0, wsem1)

    def fire_gathers(t, p):
        g = 2 * t
        pltpu.async_copy(table.at[idx_v.at[g]], bufs[p].at[pl.ds(0, GROUP)], gsems[p])
        pltpu.async_copy(table.at[idx_v.at[g + 1]], bufs[p].at[pl.ds(GROUP, GROUP)], gsems[p])

    def drain_gathers(p):
        pltpu.make_async_copy(table.at[idx_v.at[0]], bufs[p].at[pl.ds(0, GROUP)], gsems[p]).wait()
        pltpu.make_async_copy(table.at[idx_v.at[0]], bufs[p].at[pl.ds(GROUP, GROUP)], gsems[p]).wait()

    def fire_wb(t, p):
        pltpu.async_copy(bufs[p], out.at[pl.ds(base + t * SC_ROWS, SC_ROWS)], wsems[p])

    def drain_wb(p):
        pltpu.make_async_copy(bufs[p], out.at[pl.ds(base, SC_ROWS)], wsems[p]).wait()

    fire_gathers(0, 0)
    drain_gathers(0)
    fire_wb(0, 0)
    fire_gathers(1, 1)

    @pl.loop(0, (NSC - 2) // 2)
    def _(i):
        for b in range(2):
            t = 2 * i + 1 + b
            p = 1 - b
            q = b
            drain_gathers(p)
            fire_wb(t, p)
            drain_wb(q)
            fire_gathers(t + 1, q)

    drain_gathers(1)
    fire_wb(NSC - 1, 1)
    drain_wb(0)
    drain_wb(1)


def kernel(input_ids, embed_table):
    idx = input_ids.reshape(TOT // GROUP, GROUP).astype(jnp.int32)
    out = _embed_gather(embed_table, idx)
    return out.reshape(BATCH, HIST, D)
